# Initial kernel scaffold; baseline (speedup 1.0000x reference)
import jax
import jax.numpy as jnp
from jax import lax
from jax.experimental import pallas as pl
from jax.experimental.pallas import tpu as pltpu
from jax.experimental.pallas import tpu_sc as plsc

N_NODES = 10000
N_EDGES = 320000
H = 128
R = 8

NC = 2
NS = 16
NW = NC * NS
CHUNK = 128
EDGES_PER_TILE = 10112
E_PAD = EDGES_PER_TILE * NW
CHUNKS_PER_TILE = EDGES_PER_TILE // CHUNK
ACC_ROWS = 10240
ROWS_PER_TILE = ACC_ROWS // NS
DEG_W = 16


def _xw_body(wc_ref, bases_ref, h_ref, out_ref):
    r = pl.program_id(0)
    w = (wc_ref[r, 0] * bases_ref[0] + wc_ref[r, 1] * bases_ref[1]
         + wc_ref[r, 2] * bases_ref[2] + wc_ref[r, 3] * bases_ref[3])
    out_ref[0] = jnp.dot(h_ref[...], w, preferred_element_type=jnp.float32)


def _flat_body(src_ref, type_ref, out_ref):
    out_ref[...] = type_ref[...] * N_NODES + src_ref[...]


def _norm_body(pacc_ref, pdeg_ref, out_ref):
    p = pacc_ref[0, :N_NODES, :] + pacc_ref[1, :N_NODES, :]
    d = pdeg_ref[0, :N_NODES, 0:1] + pdeg_ref[1, :N_NODES, 0:1]
    out_ref[...] = p / jnp.maximum(d, 1.0)


def _sc_body(xw_hbm, flat_hbm, dst_hbm, zacc_hbm, zdeg_hbm, ones_hbm,
             pacc_hbm, pdeg_hbm,
             idx_v, dst_v, rows_v, ones_v, acc_sh, deg_sh, sem):
    c = lax.axis_index("c")
    s = lax.axis_index("s")
    wid = s * NC + c
    row0 = s * ROWS_PER_TILE
    pltpu.sync_copy(zacc_hbm, acc_sh.at[pl.ds(row0, ROWS_PER_TILE)])
    pltpu.sync_copy(zdeg_hbm, deg_sh.at[pl.ds(row0, ROWS_PER_TILE)])
    pltpu.sync_copy(ones_hbm, ones_v)
    plsc.subcore_barrier()

    base = wid * EDGES_PER_TILE

    def body(i, carry):
        off = base + i * CHUNK
        pltpu.sync_copy(flat_hbm.at[pl.ds(off, CHUNK)], idx_v)
        pltpu.sync_copy(dst_hbm.at[pl.ds(off, CHUNK)], dst_v)
        pltpu.async_copy(xw_hbm.at[idx_v], rows_v, sem).wait()
        pltpu.sync_copy(rows_v, acc_sh.at[dst_v], add=True)
        pltpu.sync_copy(ones_v, deg_sh.at[dst_v], add=True)
        return carry

    lax.fori_loop(0, CHUNKS_PER_TILE, body, 0)
    plsc.subcore_barrier()

    out_off = c * ACC_ROWS + row0
    pltpu.sync_copy(acc_sh.at[pl.ds(row0, ROWS_PER_TILE)],
                    pacc_hbm.at[pl.ds(out_off, ROWS_PER_TILE)])
    pltpu.sync_copy(deg_sh.at[pl.ds(row0, ROWS_PER_TILE)],
                    pdeg_hbm.at[pl.ds(out_off, ROWS_PER_TILE)])


def kernel(edge_index, edge_type, embed_table, bases1, w_comp1, bases2, w_comp2):
    f32 = jnp.float32
    i32 = jnp.int32
    src = edge_index[0].astype(i32)
    dst = edge_index[1].astype(i32)
    et = edge_type.astype(i32)
    pad = E_PAD - N_EDGES
    src_p = jnp.concatenate([src, jnp.zeros((pad,), i32)]).reshape(E_PAD // 128, 128)
    et_p = jnp.concatenate([et, jnp.zeros((pad,), i32)]).reshape(E_PAD // 128, 128)
    dst_p = jnp.concatenate([dst, jnp.full((pad,), N_NODES, i32)])

    flat = pl.pallas_call(
        _flat_body,
        grid=(),
        in_specs=[pl.BlockSpec((E_PAD // 128, 128), lambda: (0, 0)),
                  pl.BlockSpec((E_PAD // 128, 128), lambda: (0, 0))],
        out_specs=pl.BlockSpec((E_PAD // 128, 128), lambda: (0, 0)),
        out_shape=jax.ShapeDtypeStruct((E_PAD // 128, 128), i32),
    )(src_p, et_p).reshape(E_PAD)

    xw = pl.pallas_call(
        _xw_body,
        grid=(R,),
        in_specs=[pl.BlockSpec(memory_space=pltpu.SMEM),
                  pl.BlockSpec((4, H, H), lambda r: (0, 0, 0)),
                  pl.BlockSpec((N_NODES, H), lambda r: (0, 0))],
        out_specs=pl.BlockSpec((1, N_NODES, H), lambda r: (r, 0, 0)),
        out_shape=jax.ShapeDtypeStruct((R, N_NODES, H), f32),
    )(w_comp2, bases2, embed_table).reshape(R * N_NODES, H)

    zacc = jnp.zeros((ROWS_PER_TILE, H), f32)
    zdeg = jnp.zeros((ROWS_PER_TILE, DEG_W), f32)
    ones = jnp.ones((CHUNK, DEG_W), f32)

    mesh = plsc.VectorSubcoreMesh(core_axis_name="c", subcore_axis_name="s")
    pacc, pdeg = pl.kernel(
        _sc_body,
        out_type=(jax.ShapeDtypeStruct((NC * ACC_ROWS, H), f32),
                  jax.ShapeDtypeStruct((NC * ACC_ROWS, DEG_W), f32)),
        mesh=mesh,
        scratch_types=[
            pltpu.VMEM((CHUNK,), i32),
            pltpu.VMEM((CHUNK,), i32),
            pltpu.VMEM((CHUNK, H), f32),
            pltpu.VMEM((CHUNK, DEG_W), f32),
            pltpu.VMEM_SHARED((ACC_ROWS, H), f32),
            pltpu.VMEM_SHARED((ACC_ROWS, DEG_W), f32),
            pltpu.SemaphoreType.DMA,
        ],
    )(xw, flat, dst_p, zacc, zdeg, ones)

    pacc = pacc.reshape(NC, ACC_ROWS, H)
    pdeg = pdeg.reshape(NC, ACC_ROWS, DEG_W)

    out = pl.pallas_call(
        _norm_body,
        grid=(),
        in_specs=[pl.BlockSpec((NC, ACC_ROWS, H), lambda: (0, 0, 0)),
                  pl.BlockSpec((NC, ACC_ROWS, DEG_W), lambda: (0, 0, 0))],
        out_specs=pl.BlockSpec((N_NODES, H), lambda: (0, 0)),
        out_shape=jax.ShapeDtypeStruct((N_NODES, H), f32),
    )(pacc, pdeg)
    return out


# speedup vs baseline: 13.0889x; 13.0889x over previous
import jax
import jax.numpy as jnp
from jax import lax
from jax.experimental import pallas as pl
from jax.experimental.pallas import tpu as pltpu
from jax.experimental.pallas import tpu_sc as plsc

N_NODES = 10000
N_EDGES = 320000
H = 128
HC = H // 2
R = 8

NC = 2
NS = 16
CHUNK = 128
CHUNKS_PER_TILE = 157
EDGES_PER_TILE = CHUNKS_PER_TILE * CHUNK
E_PAD = EDGES_PER_TILE * NS
ACC_ROWS = 10240
ROWS_PER_TILE = ACC_ROWS // NS
DEG_W = 16


def _xw_body(wc_ref, bases_ref, h_ref, out_ref):
    r = pl.program_id(0)
    w = (wc_ref[r, 0] * bases_ref[0] + wc_ref[r, 1] * bases_ref[1]
         + wc_ref[r, 2] * bases_ref[2] + wc_ref[r, 3] * bases_ref[3])
    out_ref[0] = jnp.dot(h_ref[...], w, preferred_element_type=jnp.float32)


def _flat_body(src_ref, type_ref, out_ref):
    flat2 = (type_ref[...] * N_NODES + src_ref[...]) * 2
    out_ref[: E_PAD // 128] = flat2
    out_ref[E_PAD // 128 :] = flat2 + 1


def _norm_body(pacc_ref, pdeg_ref, out_ref):
    d = pdeg_ref[0, :N_NODES, 0:1] + pdeg_ref[1, :N_NODES, 0:1]
    inv = 1.0 / jnp.maximum(d, 1.0)
    out_ref[:, :HC] = pacc_ref[0, :N_NODES, :] * inv
    out_ref[:, HC:] = pacc_ref[1, :N_NODES, :] * inv


def _sc_body(xw_hbm, flat_hbm, dst_hbm, zacc_hbm, zdeg_hbm, ones_hbm,
             pacc_hbm, pdeg_hbm,
             idx_v, dst_v, rows_v, zrow_v, zdeg_v, ones_v, acc_sh, deg_sh, sem):
    c = lax.axis_index("c")
    s = lax.axis_index("s")
    row0 = s * ROWS_PER_TILE
    pltpu.sync_copy(zacc_hbm, zrow_v)
    pltpu.sync_copy(zdeg_hbm, zdeg_v)
    pltpu.sync_copy(ones_hbm, ones_v)
    for j in range(ROWS_PER_TILE // CHUNK):
        pltpu.sync_copy(zrow_v, acc_sh.at[pl.ds(row0 + j * CHUNK, CHUNK)])
        pltpu.sync_copy(zdeg_v, deg_sh.at[pl.ds(row0 + j * CHUNK, CHUNK)])
    plsc.subcore_barrier()

    base = s * EDGES_PER_TILE
    idx_base = c * E_PAD + base

    @pl.loop(0, CHUNKS_PER_TILE)
    def _edge_chunk(i):
        off = base + i * CHUNK
        pltpu.sync_copy(flat_hbm.at[pl.ds(idx_base + i * CHUNK, CHUNK)], idx_v)
        pltpu.sync_copy(dst_hbm.at[pl.ds(off, CHUNK)], dst_v)
        pltpu.async_copy(xw_hbm.at[idx_v], rows_v, sem).wait()
        pltpu.sync_copy(rows_v, acc_sh.at[dst_v], add=True)

        @pl.when(lax.rem(i, 2) == c)
        def _deg():
            pltpu.sync_copy(ones_v, deg_sh.at[dst_v], add=True)

    plsc.subcore_barrier()

    out_off = c * ACC_ROWS + row0
    for j in range(ROWS_PER_TILE // CHUNK):
        pltpu.sync_copy(acc_sh.at[pl.ds(row0 + j * CHUNK, CHUNK)], rows_v)
        pltpu.sync_copy(rows_v, pacc_hbm.at[pl.ds(out_off + j * CHUNK, CHUNK)])
        pltpu.sync_copy(deg_sh.at[pl.ds(row0 + j * CHUNK, CHUNK)], zdeg_v)
        pltpu.sync_copy(zdeg_v, pdeg_hbm.at[pl.ds(out_off + j * CHUNK, CHUNK)])


def kernel(edge_index, edge_type, embed_table, bases1, w_comp1, bases2, w_comp2):
    f32 = jnp.float32
    i32 = jnp.int32
    src = edge_index[0].astype(i32)
    dst = edge_index[1].astype(i32)
    et = edge_type.astype(i32)
    pad = E_PAD - N_EDGES
    src_p = jnp.concatenate([src, jnp.zeros((pad,), i32)]).reshape(E_PAD // 128, 128)
    et_p = jnp.concatenate([et, jnp.zeros((pad,), i32)]).reshape(E_PAD // 128, 128)
    dst_p = jnp.concatenate([dst, jnp.full((pad,), N_NODES, i32)])

    flat2 = pl.pallas_call(
        _flat_body,
        grid=(),
        in_specs=[pl.BlockSpec((E_PAD // 128, 128), lambda: (0, 0)),
                  pl.BlockSpec((E_PAD // 128, 128), lambda: (0, 0))],
        out_specs=pl.BlockSpec((NC * E_PAD // 128, 128), lambda: (0, 0)),
        out_shape=jax.ShapeDtypeStruct((NC * E_PAD // 128, 128), i32),
    )(src_p, et_p).reshape(NC * E_PAD)

    xw = pl.pallas_call(
        _xw_body,
        grid=(R,),
        in_specs=[pl.BlockSpec(memory_space=pltpu.SMEM),
                  pl.BlockSpec((4, H, H), lambda r: (0, 0, 0)),
                  pl.BlockSpec((N_NODES, H), lambda r: (0, 0))],
        out_specs=pl.BlockSpec((1, N_NODES, H), lambda r: (r, 0, 0)),
        out_shape=jax.ShapeDtypeStruct((R, N_NODES, H), f32),
    )(w_comp2, bases2, embed_table).reshape(NC * R * N_NODES, HC)

    zacc = jnp.zeros((CHUNK, HC), f32)
    zdeg = jnp.zeros((CHUNK, DEG_W), f32)
    ones = jnp.ones((CHUNK, DEG_W), f32)

    mesh = plsc.VectorSubcoreMesh(core_axis_name="c", subcore_axis_name="s")
    pacc, pdeg = pl.kernel(
        _sc_body,
        out_type=(jax.ShapeDtypeStruct((NC * ACC_ROWS, HC), f32),
                  jax.ShapeDtypeStruct((NC * ACC_ROWS, DEG_W), f32)),
        mesh=mesh,
        compiler_params=pltpu.CompilerParams(use_tc_tiling_on_sc=False),
        scratch_types=[
            pltpu.VMEM((CHUNK,), i32),
            pltpu.VMEM((CHUNK,), i32),
            pltpu.VMEM((CHUNK, HC), f32),
            pltpu.VMEM((CHUNK, HC), f32),
            pltpu.VMEM((CHUNK, DEG_W), f32),
            pltpu.VMEM((CHUNK, DEG_W), f32),
            pltpu.VMEM_SHARED((ACC_ROWS, HC), f32),
            pltpu.VMEM_SHARED((ACC_ROWS, DEG_W), f32),
            pltpu.SemaphoreType.DMA,
        ],
    )(xw, flat2, dst_p, zacc, zdeg, ones)

    pacc = pacc.reshape(NC, ACC_ROWS, HC)
    pdeg = pdeg.reshape(NC, ACC_ROWS, DEG_W)

    out = pl.pallas_call(
        _norm_body,
        grid=(),
        in_specs=[pl.BlockSpec((NC, ACC_ROWS, HC), lambda: (0, 0, 0)),
                  pl.BlockSpec((NC, ACC_ROWS, DEG_W), lambda: (0, 0, 0))],
        out_specs=pl.BlockSpec((N_NODES, H), lambda: (0, 0)),
        out_shape=jax.ShapeDtypeStruct((N_NODES, H), f32),
    )(pacc, pdeg)
    return out


# speedup vs baseline: 23.4542x; 1.7919x over previous
import jax
import jax.numpy as jnp
from jax import lax
from jax.experimental import pallas as pl
from jax.experimental.pallas import tpu as pltpu
from jax.experimental.pallas import tpu_sc as plsc

N_NODES = 10000
N_EDGES = 320000
H = 128
HC = H // 2
R = 8

NC = 2
NS = 16
CHUNK = 128
CHUNKS_PER_TILE = 157
EDGES_PER_TILE = CHUNKS_PER_TILE * CHUNK
E_PAD = EDGES_PER_TILE * NS
ACC_ROWS = 10240
ROWS_PER_TILE = ACC_ROWS // NS
DEG_W = 16


def _xw_body(wc_ref, bases_ref, h_ref, out_ref):
    r = pl.program_id(0)
    w = (wc_ref[r, 0] * bases_ref[0] + wc_ref[r, 1] * bases_ref[1]
         + wc_ref[r, 2] * bases_ref[2] + wc_ref[r, 3] * bases_ref[3])
    out_ref[0] = jnp.dot(h_ref[...], w, preferred_element_type=jnp.float32)


def _flat_body(src_ref, type_ref, out_ref):
    flat2 = (type_ref[...] * N_NODES + src_ref[...]) * 2
    out_ref[: E_PAD // 128] = flat2
    out_ref[E_PAD // 128 :] = flat2 + 1


def _norm_body(pacc_ref, pdeg_ref, out_ref):
    d = pdeg_ref[0, :N_NODES, 0:1] + pdeg_ref[1, :N_NODES, 0:1]
    inv = 1.0 / jnp.maximum(d, 1.0)
    out_ref[:, :HC] = pacc_ref[0, :N_NODES, :] * inv
    out_ref[:, HC:] = pacc_ref[1, :N_NODES, :] * inv


def _sc_body(xw_hbm, flat_hbm, dst_hbm, zacc_hbm, zdeg_hbm, ones_hbm,
             pacc_hbm, pdeg_hbm,
             fidx_v, dstx_v, rows_a, rows_b, zrow_v, zdeg_v, ones_v,
             acc_sh, deg_sh, sem_a, sem_b):
    c = lax.axis_index("c")
    s = lax.axis_index("s")
    row0 = s * ROWS_PER_TILE
    pltpu.sync_copy(flat_hbm.at[c, s], fidx_v)
    pltpu.sync_copy(dst_hbm.at[s], dstx_v)
    pltpu.sync_copy(zacc_hbm, zrow_v)
    pltpu.sync_copy(zdeg_hbm, zdeg_v)
    pltpu.sync_copy(ones_hbm, ones_v)
    for j in range(ROWS_PER_TILE // CHUNK):
        pltpu.sync_copy(zrow_v, acc_sh.at[pl.ds(row0 + j * CHUNK, CHUNK)])
        pltpu.sync_copy(zdeg_v, deg_sh.at[pl.ds(row0 + j * CHUNK, CHUNK)])
    plsc.subcore_barrier()

    pltpu.async_copy(xw_hbm.at[fidx_v.at[0]], rows_a, sem_a)
    pltpu.async_copy(xw_hbm.at[fidx_v.at[1]], rows_b, sem_b)

    @pl.loop(0, CHUNKS_PER_TILE, step=2)
    def _pair(i):
        pltpu.make_async_copy(xw_hbm.at[fidx_v.at[i]], rows_a, sem_a).wait()
        pltpu.sync_copy(rows_a, acc_sh.at[dstx_v.at[i]], add=True)

        @pl.when(c == 0)
        def _deg_a():
            pltpu.sync_copy(ones_v, deg_sh.at[dstx_v.at[i]], add=True)

        @pl.when(i + 2 < CHUNKS_PER_TILE)
        def _next_a():
            pltpu.async_copy(xw_hbm.at[fidx_v.at[i + 2]], rows_a, sem_a)

        @pl.when(i + 1 < CHUNKS_PER_TILE)
        def _b_part():
            pltpu.make_async_copy(xw_hbm.at[fidx_v.at[i + 1]], rows_b, sem_b).wait()
            pltpu.sync_copy(rows_b, acc_sh.at[dstx_v.at[i + 1]], add=True)

            @pl.when(c == 1)
            def _deg_b():
                pltpu.sync_copy(ones_v, deg_sh.at[dstx_v.at[i + 1]], add=True)

            @pl.when(i + 3 < CHUNKS_PER_TILE)
            def _next_b():
                pltpu.async_copy(xw_hbm.at[fidx_v.at[i + 3]], rows_b, sem_b)

    plsc.subcore_barrier()

    out_off = c * ACC_ROWS + row0
    for j in range(ROWS_PER_TILE // CHUNK):
        pltpu.sync_copy(acc_sh.at[pl.ds(row0 + j * CHUNK, CHUNK)], rows_a)
        pltpu.sync_copy(rows_a, pacc_hbm.at[pl.ds(out_off + j * CHUNK, CHUNK)])
        pltpu.sync_copy(deg_sh.at[pl.ds(row0 + j * CHUNK, CHUNK)], zdeg_v)
        pltpu.sync_copy(zdeg_v, pdeg_hbm.at[pl.ds(out_off + j * CHUNK, CHUNK)])


def kernel(edge_index, edge_type, embed_table, bases1, w_comp1, bases2, w_comp2):
    f32 = jnp.float32
    i32 = jnp.int32
    src = edge_index[0].astype(i32)
    dst = edge_index[1].astype(i32)
    et = edge_type.astype(i32)
    pad = E_PAD - N_EDGES
    src_p = jnp.concatenate([src, jnp.zeros((pad,), i32)]).reshape(E_PAD // 128, 128)
    et_p = jnp.concatenate([et, jnp.zeros((pad,), i32)]).reshape(E_PAD // 128, 128)
    dst_p = jnp.concatenate([dst, jnp.full((pad,), N_NODES, i32)]).reshape(
        NS, CHUNKS_PER_TILE, CHUNK)

    flat2 = pl.pallas_call(
        _flat_body,
        grid=(),
        in_specs=[pl.BlockSpec((E_PAD // 128, 128), lambda: (0, 0)),
                  pl.BlockSpec((E_PAD // 128, 128), lambda: (0, 0))],
        out_specs=pl.BlockSpec((NC * E_PAD // 128, 128), lambda: (0, 0)),
        out_shape=jax.ShapeDtypeStruct((NC * E_PAD // 128, 128), i32),
    )(src_p, et_p).reshape(NC, NS, CHUNKS_PER_TILE, CHUNK)

    xw = pl.pallas_call(
        _xw_body,
        grid=(R,),
        in_specs=[pl.BlockSpec(memory_space=pltpu.SMEM),
                  pl.BlockSpec((4, H, H), lambda r: (0, 0, 0)),
                  pl.BlockSpec((N_NODES, H), lambda r: (0, 0))],
        out_specs=pl.BlockSpec((1, N_NODES, H), lambda r: (r, 0, 0)),
        out_shape=jax.ShapeDtypeStruct((R, N_NODES, H), f32),
    )(w_comp2, bases2, embed_table).reshape(NC * R * N_NODES, HC)

    zacc = jnp.zeros((CHUNK, HC), f32)
    zdeg = jnp.zeros((CHUNK, DEG_W), f32)
    ones = jnp.ones((CHUNK, DEG_W), f32)

    mesh = plsc.VectorSubcoreMesh(core_axis_name="c", subcore_axis_name="s")
    pacc, pdeg = pl.kernel(
        _sc_body,
        out_type=(jax.ShapeDtypeStruct((NC * ACC_ROWS, HC), f32),
                  jax.ShapeDtypeStruct((NC * ACC_ROWS, DEG_W), f32)),
        mesh=mesh,
        compiler_params=pltpu.CompilerParams(use_tc_tiling_on_sc=False),
        scratch_types=[
            pltpu.VMEM((CHUNKS_PER_TILE, CHUNK), i32),
            pltpu.VMEM((CHUNKS_PER_TILE, CHUNK), i32),
            pltpu.VMEM((CHUNK, HC), f32),
            pltpu.VMEM((CHUNK, HC), f32),
            pltpu.VMEM((CHUNK, HC), f32),
            pltpu.VMEM((CHUNK, DEG_W), f32),
            pltpu.VMEM((CHUNK, DEG_W), f32),
            pltpu.VMEM_SHARED((ACC_ROWS, HC), f32),
            pltpu.VMEM_SHARED((ACC_ROWS, DEG_W), f32),
            pltpu.SemaphoreType.DMA,
            pltpu.SemaphoreType.DMA,
        ],
    )(xw, flat2, dst_p, zacc, zdeg, ones)

    pacc = pacc.reshape(NC, ACC_ROWS, HC)
    pdeg = pdeg.reshape(NC, ACC_ROWS, DEG_W)

    out = pl.pallas_call(
        _norm_body,
        grid=(),
        in_specs=[pl.BlockSpec((NC, ACC_ROWS, HC), lambda: (0, 0, 0)),
                  pl.BlockSpec((NC, ACC_ROWS, DEG_W), lambda: (0, 0, 0))],
        out_specs=pl.BlockSpec((N_NODES, H), lambda: (0, 0)),
        out_shape=jax.ShapeDtypeStruct((N_NODES, H), f32),
    )(pacc, pdeg)
    return out


# speedup vs baseline: 28.1434x; 1.1999x over previous
import jax
import jax.numpy as jnp
from jax import lax
from jax.experimental import pallas as pl
from jax.experimental.pallas import tpu as pltpu
from jax.experimental.pallas import tpu_sc as plsc

N_NODES = 10000
N_EDGES = 320000
H = 128
HC = H // 2
R = 8

NC = 2
NS = 16
CHUNK = 128
CHUNKS_PER_TILE = 157
EDGES_PER_TILE = CHUNKS_PER_TILE * CHUNK
E_PAD = EDGES_PER_TILE * NS
ACC_ROWS = 10240
ROWS_PER_TILE = ACC_ROWS // NS
DEG_W = 16


def _xw_body(wc_ref, bases_ref, h_ref, out_ref):
    r = pl.program_id(0)
    w = (wc_ref[r, 0] * bases_ref[0] + wc_ref[r, 1] * bases_ref[1]
         + wc_ref[r, 2] * bases_ref[2] + wc_ref[r, 3] * bases_ref[3])
    out_ref[0] = jnp.dot(h_ref[...], w, preferred_element_type=jnp.float32)


def _flat_body(src_ref, type_ref, out_ref):
    flat2 = (type_ref[...] * N_NODES + src_ref[...]) * 2
    out_ref[: E_PAD // 128] = flat2
    out_ref[E_PAD // 128 :] = flat2 + 1


def _norm_body(pacc_ref, pdeg_ref, out_ref):
    d = jnp.sum(pdeg_ref[:, :N_NODES], axis=0)[:, None]
    inv = 1.0 / jnp.maximum(d, 1.0)
    out_ref[:, :HC] = pacc_ref[0, :N_NODES, :] * inv
    out_ref[:, HC:] = pacc_ref[1, :N_NODES, :] * inv


def _sc_body(xw_hbm, flat_hbm, dst_hbm, zacc_hbm, zdeg_hbm,
             pacc_hbm, pdeg_hbm,
             fidx_v, dstx_v, rows_a, rows_b, rows_c, rows_d,
             deg_local, acc_sh, sem_a, sem_b, sem_c, sem_d):
    c = lax.axis_index("c")
    s = lax.axis_index("s")
    row0 = s * ROWS_PER_TILE
    bufs = (rows_a, rows_b, rows_c, rows_d)
    sems = (sem_a, sem_b, sem_c, sem_d)
    nbuf = len(bufs)
    ones16 = jnp.full((16,), 1.0, jnp.float32)
    cp_f = pltpu.async_copy(flat_hbm.at[c, s], fidx_v, sem_a)
    cp_d = pltpu.async_copy(dst_hbm.at[s], dstx_v, sem_b)
    pltpu.sync_copy(zacc_hbm, rows_a)
    pltpu.sync_copy(zdeg_hbm, deg_local)
    for j in range(ROWS_PER_TILE // CHUNK):
        pltpu.sync_copy(rows_a, acc_sh.at[pl.ds(row0 + j * CHUNK, CHUNK)])
    cp_f.wait()
    cp_d.wait()
    plsc.subcore_barrier()

    for k in range(nbuf):
        pltpu.async_copy(xw_hbm.at[fidx_v.at[k]], bufs[k], sems[k])

    @pl.loop(0, CHUNKS_PER_TILE, step=nbuf)
    def _quad(i):
        for k in range(nbuf):
            def _part(k=k):
                j = i + k
                buf, sem = bufs[k], sems[k]
                pltpu.make_async_copy(xw_hbm.at[fidx_v.at[j]], buf, sem).wait()
                pltpu.sync_copy(buf, acc_sh.at[dstx_v.at[j]], add=True)

                @pl.when(c == (k % 2))
                def _deg():
                    for l in range(CHUNK // 16):
                        idx16 = dstx_v[j, pl.ds(l * 16, 16)]
                        plsc.addupdate_scatter(deg_local, [idx16], ones16)

                @pl.when(j + nbuf < CHUNKS_PER_TILE)
                def _next():
                    pltpu.async_copy(xw_hbm.at[fidx_v.at[j + nbuf]], buf, sem)

            if k == 0:
                _part()
            else:
                pl.when(i + k < CHUNKS_PER_TILE)(_part)

    plsc.subcore_barrier()

    out_off = c * ACC_ROWS + row0
    for j in range(ROWS_PER_TILE // CHUNK):
        pltpu.sync_copy(acc_sh.at[pl.ds(row0 + j * CHUNK, CHUNK)], bufs[j % nbuf])
        pltpu.sync_copy(bufs[j % nbuf], pacc_hbm.at[pl.ds(out_off + j * CHUNK, CHUNK)])
    pltpu.sync_copy(deg_local, pdeg_hbm.at[c * NS + s])


def kernel(edge_index, edge_type, embed_table, bases1, w_comp1, bases2, w_comp2):
    f32 = jnp.float32
    i32 = jnp.int32
    src = edge_index[0].astype(i32)
    dst = edge_index[1].astype(i32)
    et = edge_type.astype(i32)
    pad = E_PAD - N_EDGES
    src_p = jnp.concatenate([src, jnp.zeros((pad,), i32)]).reshape(E_PAD // 128, 128)
    et_p = jnp.concatenate([et, jnp.zeros((pad,), i32)]).reshape(E_PAD // 128, 128)
    dst_p = jnp.concatenate([dst, jnp.full((pad,), N_NODES, i32)]).reshape(
        NS, CHUNKS_PER_TILE, CHUNK)

    flat2 = pl.pallas_call(
        _flat_body,
        grid=(),
        in_specs=[pl.BlockSpec((E_PAD // 128, 128), lambda: (0, 0)),
                  pl.BlockSpec((E_PAD // 128, 128), lambda: (0, 0))],
        out_specs=pl.BlockSpec((NC * E_PAD // 128, 128), lambda: (0, 0)),
        out_shape=jax.ShapeDtypeStruct((NC * E_PAD // 128, 128), i32),
    )(src_p, et_p).reshape(NC, NS, CHUNKS_PER_TILE, CHUNK)

    xw = pl.pallas_call(
        _xw_body,
        grid=(R,),
        in_specs=[pl.BlockSpec(memory_space=pltpu.SMEM),
                  pl.BlockSpec((4, H, H), lambda r: (0, 0, 0)),
                  pl.BlockSpec((N_NODES, H), lambda r: (0, 0))],
        out_specs=pl.BlockSpec((1, N_NODES, H), lambda r: (r, 0, 0)),
        out_shape=jax.ShapeDtypeStruct((R, N_NODES, H), f32),
    )(w_comp2, bases2, embed_table).reshape(NC * R * N_NODES, HC)

    zacc = jnp.zeros((CHUNK, HC), f32)
    zdeg = jnp.zeros((ACC_ROWS,), f32)

    mesh = plsc.VectorSubcoreMesh(core_axis_name="c", subcore_axis_name="s")
    pacc, pdeg = pl.kernel(
        _sc_body,
        out_type=(jax.ShapeDtypeStruct((NC * ACC_ROWS, HC), f32),
                  jax.ShapeDtypeStruct((NC * NS, ACC_ROWS), f32)),
        mesh=mesh,
        compiler_params=pltpu.CompilerParams(use_tc_tiling_on_sc=False,
                                             needs_layout_passes=False),
        scratch_types=[
            pltpu.VMEM((CHUNKS_PER_TILE, CHUNK), i32),
            pltpu.VMEM((CHUNKS_PER_TILE, CHUNK), i32),
            pltpu.VMEM((CHUNK, HC), f32),
            pltpu.VMEM((CHUNK, HC), f32),
            pltpu.VMEM((CHUNK, HC), f32),
            pltpu.VMEM((CHUNK, HC), f32),
            pltpu.VMEM((ACC_ROWS,), f32),
            pltpu.VMEM_SHARED((ACC_ROWS, HC), f32),
            pltpu.SemaphoreType.DMA,
            pltpu.SemaphoreType.DMA,
            pltpu.SemaphoreType.DMA,
            pltpu.SemaphoreType.DMA,
        ],
    )(xw, flat2, dst_p, zacc, zdeg)

    pacc = pacc.reshape(NC, ACC_ROWS, HC)

    out = pl.pallas_call(
        _norm_body,
        grid=(),
        in_specs=[pl.BlockSpec((NC, ACC_ROWS, HC), lambda: (0, 0, 0)),
                  pl.BlockSpec((NC * NS, ACC_ROWS), lambda: (0, 0))],
        out_specs=pl.BlockSpec((N_NODES, H), lambda: (0, 0)),
        out_shape=jax.ShapeDtypeStruct((N_NODES, H), f32),
    )(pacc, pdeg)
    return out
